# Initial kernel scaffold; baseline (speedup 1.0000x reference)
import jax
import jax.numpy as jnp
from jax.experimental import pallas as pl


def kernel(x, edge_index, batch, W_l, b_l, W_r, W_res, b_res, W1, b1, W2, b2):
    raise NotImplementedError("write your pallas kernel here")


# speedup vs baseline: 3.8315x; 3.8315x over previous
import dataclasses
import functools

import jax
import jax.numpy as jnp
from jax import lax
from jax.experimental import pallas as pl
from jax.experimental.pallas import tpu as pltpu
from jax.experimental.pallas import tpu_sc as plsc

N = 10000
E = 320000
D = 128
H = 128
G = 64
HID = 128
OUT = 2

NC = 2
NS = 16
NW = NC * NS
L = 16

CHUNK = 128
GRP = 8
NCH = 80
EPW = NCH * CHUNK
EPAD = EPW * NW
NPAD = 10240
RPS = NPAD // NS
EG = GRP * CHUNK

RB = 640
NB = NPAD // RB

_DOT = lax.Precision.HIGHEST


def _row_index_list():
    s = jnp.arange(NS)[:, None, None]
    z = jnp.arange(8)[None, :, None]
    lane = jnp.arange(CHUNK)[None, None, :]
    idx = s * RPS + z * CHUNK + lane
    idx = jnp.where(z < RPS // CHUNK, idx, 0)
    return idx.reshape(NS * 8, CHUNK).astype(jnp.int32)


def _sc_segment_sum(src2d, dst2d, dst1d, x, ridx, zsum, zcnt):
    mesh = plsc.VectorSubcoreMesh(
        core_axis_name="c", subcore_axis_name="s",
        num_cores=NC, num_subcores=NS)

    cp = pltpu.CompilerParams()
    if "needs_layout_passes" in pltpu.CompilerParams.__dataclass_fields__:
        cp = dataclasses.replace(cp, needs_layout_passes=False)

    @functools.partial(
        pl.kernel,
        compiler_params=cp,
        out_type=(
            jax.ShapeDtypeStruct((NC * NPAD, D), jnp.float32),
            jax.ShapeDtypeStruct((NW * NPAD,), jnp.float32),
        ),
        mesh=mesh,
        scratch_types=[
            pltpu.VMEM((GRP, CHUNK), jnp.int32),
            pltpu.VMEM((GRP, CHUNK), jnp.int32),
            pltpu.VMEM((EG,), jnp.int32),
            pltpu.VMEM((CHUNK, D), jnp.float32),
            pltpu.VMEM((NPAD,), jnp.float32),
            pltpu.VMEM_SHARED((NPAD, D), jnp.float32),
            pltpu.SemaphoreType.DMA,
        ],
    )
    def k(src_hbm, dst_hbm, dst1_hbm, x_hbm, ridx_hbm, zs_hbm, zc_hbm,
          sum_hbm, cnt_hbm,
          src_v, dst_v, dstf_v, rows_v, cnt_v, acc_sh, sem):
        cid = lax.axis_index("c")
        sid = lax.axis_index("s")
        wid = sid * NC + cid
        obase = cid * NPAD + sid * RPS
        nz = RPS // CHUNK

        pltpu.sync_copy(zs_hbm, rows_v)
        pltpu.sync_copy(zc_hbm, cnt_v)
        pltpu.sync_copy(ridx_hbm.at[pl.ds(sid * 8, GRP)], src_v)

        @pl.loop(0, nz)
        def _(z):
            pltpu.sync_copy(rows_v, acc_sh.at[src_v.at[z]])

        plsc.subcore_barrier()

        ones16 = jnp.full((L,), 1.0, jnp.float32)

        @pl.loop(0, NCH // GRP)
        def _(g):
            pltpu.sync_copy(src_hbm.at[pl.ds(wid * NCH + g * GRP, GRP)],
                            src_v)
            pltpu.sync_copy(dst_hbm.at[pl.ds(wid * NCH + g * GRP, GRP)],
                            dst_v)
            pltpu.sync_copy(dst1_hbm.at[pl.ds(wid * EPW + g * EG, EG)],
                            dstf_v)

            @pl.loop(0, GRP)
            def _(j):
                pltpu.async_copy(x_hbm.at[src_v.at[j]], rows_v, sem).wait()
                pltpu.sync_copy(rows_v, acc_sh.at[dst_v.at[j]], add=True)

            @pl.loop(0, EG // L)
            def _(t):
                idx = dstf_v[pl.ds(t * L, L)]
                plsc.addupdate_scatter(cnt_v, [idx], ones16)

        plsc.subcore_barrier()

        pltpu.sync_copy(ridx_hbm.at[pl.ds(sid * 8, GRP)], src_v)

        @pl.loop(0, nz)
        def _(z):
            pltpu.sync_copy(acc_sh.at[src_v.at[z]], rows_v)
            pltpu.sync_copy(rows_v, sum_hbm.at[pl.ds(obase + z * CHUNK, CHUNK)])

        pltpu.sync_copy(cnt_v, cnt_hbm.at[pl.ds(wid * NPAD, NPAD)])

    return k(src2d, dst2d, dst1d, x, ridx, zsum, zcnt)


def _tc_body(x_ref, s0_ref, s1_ref, cw_ref, bt_ref,
             wl_ref, wr_ref, wres_ref, bl_ref, bres_ref,
             w1_ref, b1_ref, w2_ref, b2_ref,
             o_ref, ps_ref, gc_ref):
    i = pl.program_id(0)

    @pl.when(i == 0)
    def _():
        ps_ref[...] = jnp.zeros_like(ps_ref)
        gc_ref[...] = jnp.zeros_like(gc_ref)

    summed = s0_ref[...] + s1_ref[...]
    cnt = jnp.sum(cw_ref[...], axis=0)[:, None]
    neigh = summed / jnp.maximum(cnt, 1.0)
    h = lax.dot_general(neigh, wl_ref[...], (((1,), (1,)), ((), ())),
                        precision=_DOT)
    h += lax.dot_general(x_ref[...], wr_ref[...] + wres_ref[...],
                         (((1,), (1,)), ((), ())), precision=_DOT)
    h += bl_ref[...] + bres_ref[...]
    bt = bt_ref[0]
    onehot = (lax.broadcasted_iota(jnp.int32, (G, RB), 0) == bt
              ).astype(jnp.float32)
    ps_ref[...] += lax.dot_general(onehot, h, (((1,), (0,)), ((), ())),
                                   precision=_DOT)
    gc_ref[...] += jnp.broadcast_to(
        jnp.sum(onehot, axis=1, keepdims=True), (G, D))

    @pl.when(i == NB - 1)
    def _():
        pooled = ps_ref[...] / jnp.maximum(gc_ref[...], 1.0)
        z = lax.dot_general(pooled, w1_ref[...], (((1,), (1,)), ((), ())),
                            precision=_DOT) + b1_ref[...]
        z = jnp.maximum(z, 0.0)
        o_ref[...] = lax.dot_general(z, w2_ref[...], (((1,), (1,)), ((), ())),
                                     precision=_DOT) + b2_ref[...]


def _tc_dense(xp, sums, cntw, batch_p, W_l, b_l, W_r, W_res, b_res,
              W1, b1, W2, b2):
    return pl.pallas_call(
        _tc_body,
        grid=(NB,),
        in_specs=[
            pl.BlockSpec((RB, D), lambda i: (i, 0)),
            pl.BlockSpec((RB, D), lambda i: (i, 0)),
            pl.BlockSpec((RB, D), lambda i: (NB + i, 0)),
            pl.BlockSpec((NW, RB), lambda i: (0, i)),
            pl.BlockSpec((1, 1, RB), lambda i: (i, 0, 0)),
            pl.BlockSpec((H, D), lambda i: (0, 0)),
            pl.BlockSpec((H, D), lambda i: (0, 0)),
            pl.BlockSpec((H, D), lambda i: (0, 0)),
            pl.BlockSpec((1, D), lambda i: (0, 0)),
            pl.BlockSpec((1, D), lambda i: (0, 0)),
            pl.BlockSpec((HID, H), lambda i: (0, 0)),
            pl.BlockSpec((1, HID), lambda i: (0, 0)),
            pl.BlockSpec((OUT, HID), lambda i: (0, 0)),
            pl.BlockSpec((1, OUT), lambda i: (0, 0)),
        ],
        out_specs=pl.BlockSpec((G, OUT), lambda i: (0, 0)),
        out_shape=jax.ShapeDtypeStruct((G, OUT), jnp.float32),
        scratch_shapes=[
            pltpu.VMEM((G, D), jnp.float32),
            pltpu.VMEM((G, D), jnp.float32),
        ],
    )(xp, sums, sums, cntw, batch_p, W_l, W_r, W_res,
      b_l.reshape(1, D), b_res.reshape(1, D), W1, b1.reshape(1, HID),
      W2, b2.reshape(1, OUT))


def kernel(x, edge_index, batch, W_l, b_l, W_r, W_res, b_res, W1, b1, W2, b2):
    src = edge_index[0]
    dst = edge_index[1]
    pad_e = EPAD - E
    src1d = jnp.concatenate([src, jnp.zeros((pad_e,), jnp.int32)])
    dst1d = jnp.concatenate([dst, jnp.full((pad_e,), NPAD - 1, jnp.int32)])
    src2d = src1d.reshape(NW * NCH, CHUNK)
    dst2d = dst1d.reshape(NW * NCH, CHUNK)
    zsum = jnp.zeros((CHUNK, D), jnp.float32)
    zcnt = jnp.zeros((NPAD,), jnp.float32)
    sums, cnts = _sc_segment_sum(src2d, dst2d, dst1d, x, _row_index_list(),
                                 zsum, zcnt)

    xp = jnp.pad(x, ((0, NPAD - N), (0, 0)))
    batch_p = jnp.pad(batch, (0, NPAD - N),
                      constant_values=G).reshape(NB, 1, RB)
    return _tc_dense(xp, sums, cnts.reshape(NW, NPAD), batch_p,
                     W_l, b_l, W_r, W_res, b_res, W1, b1, W2, b2)


# speedup vs baseline: 4.6927x; 1.2248x over previous
import dataclasses
import functools

import jax
import jax.numpy as jnp
from jax import lax
from jax.experimental import pallas as pl
from jax.experimental.pallas import tpu as pltpu
from jax.experimental.pallas import tpu_sc as plsc

N = 10000
E = 320000
D = 128
H = 128
G = 64
HID = 128
OUT = 2

NC = 2
NS = 16
NW = NC * NS
L = 16

CHUNK = 128
GRP = 8
NCHP = 160
NCH0 = 120
NCH1 = NCHP - NCH0
EPAD = NS * NCHP * CHUNK
NPAD = 10240
RPS = NPAD // NS
EG = GRP * CHUNK

RB = 640
NB = NPAD // RB

_DOT = lax.Precision.HIGHEST


def _row_index_list():
    s = jnp.arange(NS)[:, None, None]
    z = jnp.arange(8)[None, :, None]
    lane = jnp.arange(CHUNK)[None, None, :]
    idx = s * RPS + z * CHUNK + lane
    idx = jnp.where(z < RPS // CHUNK, idx, 0)
    return idx.reshape(NS * 8, CHUNK).astype(jnp.int32)


def _sc_segment_sum(src2d, dst2d, dst1d, x, ridx, zsum, zcnt):
    mesh = plsc.VectorSubcoreMesh(
        core_axis_name="c", subcore_axis_name="s",
        num_cores=NC, num_subcores=NS)

    cp = pltpu.CompilerParams()
    if "needs_layout_passes" in pltpu.CompilerParams.__dataclass_fields__:
        cp = dataclasses.replace(cp, needs_layout_passes=False)

    @functools.partial(
        pl.kernel,
        compiler_params=cp,
        out_type=(
            jax.ShapeDtypeStruct((NC * NPAD, D), jnp.float32),
            jax.ShapeDtypeStruct((NW * NPAD,), jnp.float32),
        ),
        mesh=mesh,
        scratch_types=[
            pltpu.VMEM((GRP, CHUNK), jnp.int32),
            pltpu.VMEM((GRP, CHUNK), jnp.int32),
            pltpu.VMEM((EG,), jnp.int32),
            pltpu.VMEM((CHUNK, D), jnp.float32),
            pltpu.VMEM((NPAD,), jnp.float32),
            pltpu.VMEM_SHARED((NPAD, D), jnp.float32),
            pltpu.SemaphoreType.DMA,
        ],
    )
    def k(src_hbm, dst_hbm, dst1_hbm, x_hbm, ridx_hbm, zs_hbm, zc_hbm,
          sum_hbm, cnt_hbm,
          src_v, dst_v, dstf_v, rows_v, cnt_v, acc_sh, sem):
        cid = lax.axis_index("c")
        sid = lax.axis_index("s")
        wid = sid * NC + cid
        obase = cid * NPAD + sid * RPS
        nz = RPS // CHUNK

        pltpu.sync_copy(zs_hbm, rows_v)
        pltpu.sync_copy(zc_hbm, cnt_v)
        pltpu.sync_copy(ridx_hbm.at[pl.ds(sid * 8, GRP)], src_v)

        @pl.loop(0, nz)
        def _(z):
            pltpu.sync_copy(rows_v, acc_sh.at[src_v.at[z]])

        plsc.subcore_barrier()

        ones16 = jnp.full((L,), 1.0, jnp.float32)

        cbase = sid * NCHP + cid * NCH0
        ng = lax.select(cid == 0, NCH0 // GRP, NCH1 // GRP)

        @pl.loop(0, ng)
        def _(g):
            pltpu.sync_copy(src_hbm.at[pl.ds(cbase + g * GRP, GRP)],
                            src_v)
            pltpu.sync_copy(dst_hbm.at[pl.ds(cbase + g * GRP, GRP)],
                            dst_v)
            pltpu.sync_copy(dst1_hbm.at[pl.ds(cbase * CHUNK + g * EG, EG)],
                            dstf_v)

            @pl.loop(0, GRP)
            def _(j):
                pltpu.async_copy(x_hbm.at[src_v.at[j]], rows_v, sem).wait()
                pltpu.sync_copy(rows_v, acc_sh.at[dst_v.at[j]], add=True)

            @pl.loop(0, EG // L)
            def _(t):
                idx = dstf_v[pl.ds(t * L, L)]
                plsc.addupdate_scatter(cnt_v, [idx], ones16)

        plsc.subcore_barrier()

        pltpu.sync_copy(ridx_hbm.at[pl.ds(sid * 8, GRP)], src_v)

        @pl.loop(0, nz)
        def _(z):
            pltpu.sync_copy(acc_sh.at[src_v.at[z]], rows_v)
            pltpu.sync_copy(rows_v, sum_hbm.at[pl.ds(obase + z * CHUNK, CHUNK)])

        pltpu.sync_copy(cnt_v, cnt_hbm.at[pl.ds(wid * NPAD, NPAD)])

    return k(src2d, dst2d, dst1d, x, ridx, zsum, zcnt)


def _tc_body(x_ref, s0_ref, s1_ref, cw_ref, bt_ref,
             wl_ref, wr_ref, wres_ref, bl_ref, bres_ref,
             w1_ref, b1_ref, w2_ref, b2_ref,
             o_ref, ps_ref, gc_ref):
    i = pl.program_id(0)

    @pl.when(i == 0)
    def _():
        ps_ref[...] = jnp.zeros_like(ps_ref)
        gc_ref[...] = jnp.zeros_like(gc_ref)

    summed = s0_ref[...] + s1_ref[...]
    cnt = jnp.sum(cw_ref[...], axis=0)[:, None]
    neigh = summed / jnp.maximum(cnt, 1.0)
    h = lax.dot_general(neigh, wl_ref[...], (((1,), (1,)), ((), ())),
                        precision=_DOT)
    h += lax.dot_general(x_ref[...], wr_ref[...] + wres_ref[...],
                         (((1,), (1,)), ((), ())), precision=_DOT)
    h += bl_ref[...] + bres_ref[...]
    bt = bt_ref[0]
    onehot = (lax.broadcasted_iota(jnp.int32, (G, RB), 0) == bt
              ).astype(jnp.float32)
    ps_ref[...] += lax.dot_general(onehot, h, (((1,), (0,)), ((), ())),
                                   precision=_DOT)
    gc_ref[...] += jnp.broadcast_to(
        jnp.sum(onehot, axis=1, keepdims=True), (G, D))

    @pl.when(i == NB - 1)
    def _():
        pooled = ps_ref[...] / jnp.maximum(gc_ref[...], 1.0)
        z = lax.dot_general(pooled, w1_ref[...], (((1,), (1,)), ((), ())),
                            precision=_DOT) + b1_ref[...]
        z = jnp.maximum(z, 0.0)
        o_ref[...] = lax.dot_general(z, w2_ref[...], (((1,), (1,)), ((), ())),
                                     precision=_DOT) + b2_ref[...]


def _tc_dense(xp, sums, cntw, batch_p, W_l, b_l, W_r, W_res, b_res,
              W1, b1, W2, b2):
    return pl.pallas_call(
        _tc_body,
        grid=(NB,),
        in_specs=[
            pl.BlockSpec((RB, D), lambda i: (i, 0)),
            pl.BlockSpec((RB, D), lambda i: (i, 0)),
            pl.BlockSpec((RB, D), lambda i: (NB + i, 0)),
            pl.BlockSpec((NW, RB), lambda i: (0, i)),
            pl.BlockSpec((1, 1, RB), lambda i: (i, 0, 0)),
            pl.BlockSpec((H, D), lambda i: (0, 0)),
            pl.BlockSpec((H, D), lambda i: (0, 0)),
            pl.BlockSpec((H, D), lambda i: (0, 0)),
            pl.BlockSpec((1, D), lambda i: (0, 0)),
            pl.BlockSpec((1, D), lambda i: (0, 0)),
            pl.BlockSpec((HID, H), lambda i: (0, 0)),
            pl.BlockSpec((1, HID), lambda i: (0, 0)),
            pl.BlockSpec((OUT, HID), lambda i: (0, 0)),
            pl.BlockSpec((1, OUT), lambda i: (0, 0)),
        ],
        out_specs=pl.BlockSpec((G, OUT), lambda i: (0, 0)),
        out_shape=jax.ShapeDtypeStruct((G, OUT), jnp.float32),
        scratch_shapes=[
            pltpu.VMEM((G, D), jnp.float32),
            pltpu.VMEM((G, D), jnp.float32),
        ],
    )(xp, sums, sums, cntw, batch_p, W_l, W_r, W_res,
      b_l.reshape(1, D), b_res.reshape(1, D), W1, b1.reshape(1, HID),
      W2, b2.reshape(1, OUT))


def kernel(x, edge_index, batch, W_l, b_l, W_r, W_res, b_res, W1, b1, W2, b2):
    src = edge_index[0]
    dst = edge_index[1]
    pad_e = EPAD - E
    src1d = jnp.concatenate([src, jnp.zeros((pad_e,), jnp.int32)])
    dst1d = jnp.concatenate([dst, jnp.full((pad_e,), NPAD - 1, jnp.int32)])
    src2d = src1d.reshape(NS * NCHP, CHUNK)
    dst2d = dst1d.reshape(NS * NCHP, CHUNK)
    zsum = jnp.zeros((CHUNK, D), jnp.float32)
    zcnt = jnp.zeros((NPAD,), jnp.float32)
    sums, cnts = _sc_segment_sum(src2d, dst2d, dst1d, x, _row_index_list(),
                                 zsum, zcnt)

    xp = jnp.pad(x, ((0, NPAD - N), (0, 0)))
    batch_p = jnp.pad(batch, (0, NPAD - N),
                      constant_values=G).reshape(NB, 1, RB)
    return _tc_dense(xp, sums, cnts.reshape(NW, NPAD), batch_p,
                     W_l, b_l, W_r, W_res, b_res, W1, b1, W2, b2)


# speedup vs baseline: 5.2306x; 1.1146x over previous
import dataclasses
import functools

import jax
import jax.numpy as jnp
from jax import lax
from jax.experimental import pallas as pl
from jax.experimental.pallas import tpu as pltpu
from jax.experimental.pallas import tpu_sc as plsc

N = 10000
E = 320000
D = 128
H = 128
G = 64
HID = 128
OUT = 2

NC = 2
NS = 16
NW = NC * NS
L = 16

CHUNK = 128
GRP = 8
NCHP = 160
NCH0 = 120
NCH1 = NCHP - NCH0
EPAD = NS * NCHP * CHUNK
NPAD = 10240
RPS = NPAD // NS
EG = GRP * CHUNK

RB = 640
NB = NPAD // RB

_DOT = lax.Precision.HIGHEST


def _row_index_list():
    s = jnp.arange(NS)[:, None, None]
    z = jnp.arange(8)[None, :, None]
    lane = jnp.arange(CHUNK)[None, None, :]
    idx = s * RPS + z * CHUNK + lane
    idx = jnp.where(z < RPS // CHUNK, idx, 0)
    return idx.reshape(NS * 8, CHUNK).astype(jnp.int32)


def _sc_segment_sum(src2d, dst2d, dst1d, x, ridx, zsum, zcnt):
    mesh = plsc.VectorSubcoreMesh(
        core_axis_name="c", subcore_axis_name="s",
        num_cores=NC, num_subcores=NS)

    cp = pltpu.CompilerParams()
    if "needs_layout_passes" in pltpu.CompilerParams.__dataclass_fields__:
        cp = dataclasses.replace(cp, needs_layout_passes=False)

    @functools.partial(
        pl.kernel,
        compiler_params=cp,
        out_type=(
            jax.ShapeDtypeStruct((NC * NPAD, D), jnp.float32),
            jax.ShapeDtypeStruct((NW * NPAD,), jnp.float32),
        ),
        mesh=mesh,
        scratch_types=[
            pltpu.VMEM((GRP, CHUNK), jnp.int32),
            pltpu.VMEM((GRP, CHUNK), jnp.int32),
            pltpu.VMEM((EG,), jnp.int32),
            pltpu.VMEM((CHUNK, D), jnp.float32),
            pltpu.VMEM((CHUNK, D), jnp.float32),
            pltpu.VMEM((NPAD,), jnp.float32),
            pltpu.VMEM_SHARED((NPAD, D), jnp.float32),
            pltpu.SemaphoreType.DMA,
            pltpu.SemaphoreType.DMA,
        ],
    )
    def k(src_hbm, dst_hbm, dst1_hbm, x_hbm, ridx_hbm, zs_hbm, zc_hbm,
          sum_hbm, cnt_hbm,
          src_v, dst_v, dstf_v, rows_v, rows2_v, cnt_v, acc_sh, sem, sem2):
        cid = lax.axis_index("c")
        sid = lax.axis_index("s")
        wid = sid * NC + cid
        obase = cid * NPAD + sid * RPS
        nz = RPS // CHUNK

        pltpu.sync_copy(zs_hbm, rows_v)
        pltpu.sync_copy(zc_hbm, cnt_v)
        pltpu.sync_copy(ridx_hbm.at[pl.ds(sid * 8, GRP)], src_v)

        @pl.loop(0, nz)
        def _(z):
            pltpu.sync_copy(rows_v, acc_sh.at[src_v.at[z]])

        plsc.subcore_barrier()

        ones16 = jnp.full((L,), 1.0, jnp.float32)

        cbase = sid * NCHP + cid * NCH0
        ng = lax.select(cid == 0, NCH0 // GRP, NCH1 // GRP)

        @pl.loop(0, ng)
        def _(g):
            pltpu.sync_copy(src_hbm.at[pl.ds(cbase + g * GRP, GRP)],
                            src_v)
            pltpu.sync_copy(dst_hbm.at[pl.ds(cbase + g * GRP, GRP)],
                            dst_v)
            pltpu.sync_copy(dst1_hbm.at[pl.ds(cbase * CHUNK + g * EG, EG)],
                            dstf_v)

            bufs = (rows_v, rows2_v)
            sems = (sem, sem2)
            descs = [pltpu.async_copy(x_hbm.at[src_v.at[0]], rows_v, sem)]
            for j in range(GRP):
                if j + 1 < GRP:
                    descs.append(pltpu.async_copy(
                        x_hbm.at[src_v.at[j + 1]],
                        bufs[(j + 1) % 2], sems[(j + 1) % 2]))
                descs[j].wait()
                pltpu.sync_copy(bufs[j % 2], acc_sh.at[dst_v.at[j]], add=True)

            @pl.loop(0, EG // L)
            def _(t):
                idx = dstf_v[pl.ds(t * L, L)]
                plsc.addupdate_scatter(cnt_v, [idx], ones16)

        plsc.subcore_barrier()

        pltpu.sync_copy(ridx_hbm.at[pl.ds(sid * 8, GRP)], src_v)

        @pl.loop(0, nz)
        def _(z):
            pltpu.sync_copy(acc_sh.at[src_v.at[z]], rows_v)
            pltpu.sync_copy(rows_v, sum_hbm.at[pl.ds(obase + z * CHUNK, CHUNK)])

        pltpu.sync_copy(cnt_v, cnt_hbm.at[pl.ds(wid * NPAD, NPAD)])

    return k(src2d, dst2d, dst1d, x, ridx, zsum, zcnt)


def _tc_body(x_ref, s0_ref, s1_ref, cw_ref, bt_ref,
             wl_ref, wr_ref, wres_ref, bl_ref, bres_ref,
             w1_ref, b1_ref, w2_ref, b2_ref,
             o_ref, ps_ref, gc_ref):
    i = pl.program_id(0)

    @pl.when(i == 0)
    def _():
        ps_ref[...] = jnp.zeros_like(ps_ref)
        gc_ref[...] = jnp.zeros_like(gc_ref)

    summed = s0_ref[...] + s1_ref[...]
    cnt = jnp.sum(cw_ref[...], axis=0)[:, None]
    neigh = summed / jnp.maximum(cnt, 1.0)
    h = lax.dot_general(neigh, wl_ref[...], (((1,), (1,)), ((), ())),
                        precision=_DOT)
    h += lax.dot_general(x_ref[...], wr_ref[...] + wres_ref[...],
                         (((1,), (1,)), ((), ())), precision=_DOT)
    h += bl_ref[...] + bres_ref[...]
    bt = bt_ref[0]
    onehot = (lax.broadcasted_iota(jnp.int32, (G, RB), 0) == bt
              ).astype(jnp.float32)
    ps_ref[...] += lax.dot_general(onehot, h, (((1,), (0,)), ((), ())),
                                   precision=_DOT)
    gc_ref[...] += jnp.broadcast_to(
        jnp.sum(onehot, axis=1, keepdims=True), (G, D))

    @pl.when(i == NB - 1)
    def _():
        pooled = ps_ref[...] / jnp.maximum(gc_ref[...], 1.0)
        z = lax.dot_general(pooled, w1_ref[...], (((1,), (1,)), ((), ())),
                            precision=_DOT) + b1_ref[...]
        z = jnp.maximum(z, 0.0)
        o_ref[...] = lax.dot_general(z, w2_ref[...], (((1,), (1,)), ((), ())),
                                     precision=_DOT) + b2_ref[...]


def _tc_dense(xp, sums, cntw, batch_p, W_l, b_l, W_r, W_res, b_res,
              W1, b1, W2, b2):
    return pl.pallas_call(
        _tc_body,
        grid=(NB,),
        in_specs=[
            pl.BlockSpec((RB, D), lambda i: (i, 0)),
            pl.BlockSpec((RB, D), lambda i: (i, 0)),
            pl.BlockSpec((RB, D), lambda i: (NB + i, 0)),
            pl.BlockSpec((NW, RB), lambda i: (0, i)),
            pl.BlockSpec((1, 1, RB), lambda i: (i, 0, 0)),
            pl.BlockSpec((H, D), lambda i: (0, 0)),
            pl.BlockSpec((H, D), lambda i: (0, 0)),
            pl.BlockSpec((H, D), lambda i: (0, 0)),
            pl.BlockSpec((1, D), lambda i: (0, 0)),
            pl.BlockSpec((1, D), lambda i: (0, 0)),
            pl.BlockSpec((HID, H), lambda i: (0, 0)),
            pl.BlockSpec((1, HID), lambda i: (0, 0)),
            pl.BlockSpec((OUT, HID), lambda i: (0, 0)),
            pl.BlockSpec((1, OUT), lambda i: (0, 0)),
        ],
        out_specs=pl.BlockSpec((G, OUT), lambda i: (0, 0)),
        out_shape=jax.ShapeDtypeStruct((G, OUT), jnp.float32),
        scratch_shapes=[
            pltpu.VMEM((G, D), jnp.float32),
            pltpu.VMEM((G, D), jnp.float32),
        ],
    )(xp, sums, sums, cntw, batch_p, W_l, W_r, W_res,
      b_l.reshape(1, D), b_res.reshape(1, D), W1, b1.reshape(1, HID),
      W2, b2.reshape(1, OUT))


def kernel(x, edge_index, batch, W_l, b_l, W_r, W_res, b_res, W1, b1, W2, b2):
    src = edge_index[0]
    dst = edge_index[1]
    pad_e = EPAD - E
    src1d = jnp.concatenate([src, jnp.zeros((pad_e,), jnp.int32)])
    dst1d = jnp.concatenate([dst, jnp.full((pad_e,), NPAD - 1, jnp.int32)])
    src2d = src1d.reshape(NS * NCHP, CHUNK)
    dst2d = dst1d.reshape(NS * NCHP, CHUNK)
    zsum = jnp.zeros((CHUNK, D), jnp.float32)
    zcnt = jnp.zeros((NPAD,), jnp.float32)
    sums, cnts = _sc_segment_sum(src2d, dst2d, dst1d, x, _row_index_list(),
                                 zsum, zcnt)

    xp = jnp.pad(x, ((0, NPAD - N), (0, 0)))
    batch_p = jnp.pad(batch, (0, NPAD - N),
                      constant_values=G).reshape(NB, 1, RB)
    return _tc_dense(xp, sums, cnts.reshape(NW, NPAD), batch_p,
                     W_l, b_l, W_r, W_res, b_res, W1, b1, W2, b2)
